# baseline (speedup 1.0000x reference)
import jax
import jax.numpy as jnp
from jax import lax
from jax.experimental import pallas as pl
from jax.experimental.pallas import tpu as pltpu
from jax.experimental.pallas import tpu_sc as plsc

EMBED_DIM = 18
PAD_DIM = 24
FIELDS = 26
BATCH = 4096
VOCAB = 1000000
IN_DIM = FIELDS * EMBED_DIM
IN_DIM_P = FIELDS * PAD_DIM

NUM_CORES = 2
NUM_SUBCORES = 16
NW = NUM_CORES * NUM_SUBCORES
ROWS = BATCH * FIELDS
RPW = ROWS // NW

NSLAB = (VOCAB + 127) // 128
SPW = (NSLAB + NW - 1) // NW
LIN_ROWS = NSLAB * 128
SLAB_WORDS = 128 * PAD_DIM
LIN_LEN = NSLAB * SLAB_WORDS


def _detr_body(tab_hbm, out_hbm, slab0, slab1, lin0, lin1, sem_i0, sem_i1, sem_o):
    wid = lax.axis_index("s") * NUM_CORES + lax.axis_index("c")
    d1 = lax.iota(jnp.int32, 16)
    d2 = jnp.minimum(d1 + 8, EMBED_DIM - 1)

    def cidx(s):
        return jnp.minimum(s * NW + wid, NSLAB - 1)

    def start_in(buf, sem, s):
        c = cidx(s)
        pltpu.async_copy(
            tab_hbm.at[:, pl.ds(pl.multiple_of(c * 128, 128), 128)], buf, sem)

    def wait_in(buf, sem):
        pltpu.make_async_copy(tab_hbm.at[:, pl.ds(0, 128)], buf, sem).wait()

    def drain_out(lin):
        pltpu.make_async_copy(out_hbm.at[pl.ds(0, SLAB_WORDS)], lin, sem_o).wait()

    def transpose(buf, lin):
        for l in range(128):
            ls = jnp.full((16,), l, jnp.int32)
            v1 = plsc.load_gather(buf, [d1, ls])
            v2 = plsc.load_gather(buf, [d2, ls])
            lin[pl.ds(l * PAD_DIM, 16)] = v1
            lin[pl.ds(l * PAD_DIM + 8, 16)] = v2

    def start_out(lin, s):
        c = cidx(s)
        pltpu.async_copy(
            lin,
            out_hbm.at[pl.ds(pl.multiple_of(c * SLAB_WORDS, 1024), SLAB_WORDS)],
            sem_o)

    start_in(slab0, sem_i0, 0)

    def body(j, carry):
        s0 = 2 * j
        wait_in(slab0, sem_i0)
        start_in(slab1, sem_i1, s0 + 1)

        @pl.when(j > 0)
        def _():
            drain_out(lin0)

        transpose(slab0, lin0)
        start_out(lin0, s0)

        wait_in(slab1, sem_i1)
        start_in(slab0, sem_i0, s0 + 2)

        @pl.when(j > 0)
        def _():
            drain_out(lin1)

        transpose(slab1, lin1)
        start_out(lin1, s0 + 1)
        return carry

    lax.fori_loop(0, (SPW - 1) // 2, body, 0)
    wait_in(slab0, sem_i0)
    drain_out(lin0)
    transpose(slab0, lin0)
    start_out(lin0, SPW - 1)
    drain_out(lin1)
    drain_out(lin0)


_detranspose = pl.kernel(
    _detr_body,
    out_type=jax.ShapeDtypeStruct((LIN_LEN,), jnp.float32),
    mesh=plsc.VectorSubcoreMesh(core_axis_name="c", subcore_axis_name="s"),
    scratch_types=[
        pltpu.VMEM((EMBED_DIM, 128), jnp.float32),
        pltpu.VMEM((EMBED_DIM, 128), jnp.float32),
        pltpu.VMEM((SLAB_WORDS,), jnp.float32),
        pltpu.VMEM((SLAB_WORDS,), jnp.float32),
        pltpu.SemaphoreType.DMA,
        pltpu.SemaphoreType.DMA,
        pltpu.SemaphoreType.DMA,
    ],
    compiler_params=pltpu.CompilerParams(needs_layout_passes=False),
)


def _gather_body(idx_hbm, lin_hbm, out_hbm, idx_v, rows_v, sem):
    wid = lax.axis_index("s") * NUM_CORES + lax.axis_index("c")
    base = wid * RPW
    pltpu.sync_copy(idx_hbm.at[pl.ds(base, RPW)], idx_v)
    pltpu.async_copy(lin_hbm.at[idx_v], rows_v, sem).wait()
    pltpu.sync_copy(rows_v, out_hbm.at[pl.ds(base, RPW)])


_gather = pl.kernel(
    _gather_body,
    out_type=jax.ShapeDtypeStruct((ROWS, PAD_DIM), jnp.float32),
    mesh=plsc.VectorSubcoreMesh(core_axis_name="c", subcore_axis_name="s"),
    scratch_types=[
        pltpu.VMEM((RPW,), jnp.int32),
        pltpu.VMEM((RPW, PAD_DIM), jnp.float32),
        pltpu.SemaphoreType.DMA,
    ],
    compiler_params=pltpu.CompilerParams(use_tc_tiling_on_sc=False),
)

BB = 1024


def _mlp_body(feat_ref,
              cW0, cb0, cW1, cb1, cW2, cb2, cW3, cb3, cW4, cb4,
              vW0, vb0, vW1, vb1, vW2, vb2, vW3, vb3, vW4, vb4,
              out_ref):
    f = feat_ref[...]

    def tower(Ws, bs):
        h = f
        for i in range(4):
            h = jnp.dot(h, Ws[i][...], preferred_element_type=jnp.float32)
            h = jnp.maximum(h + bs[i][...], 0.0)
        h = jnp.dot(h, Ws[4][...], preferred_element_type=jnp.float32)
        return h + bs[4][...]

    ctr = tower([cW0, cW1, cW2, cW3, cW4], [cb0, cb1, cb2, cb3, cb4])
    cvr = tower([vW0, vW1, vW2, vW3, vW4], [vb0, vb1, vb2, vb3, vb4])
    both = jnp.concatenate([ctr, cvr], axis=1)
    out_ref[...] = 1.0 / (1.0 + jnp.exp(-both))


def _mlp(feat, weights):
    def w_spec(w):
        return pl.BlockSpec(w.shape, lambda i: (0,) * w.ndim)

    in_specs = [pl.BlockSpec((BB, IN_DIM_P), lambda i: (i, 0))]
    in_specs += [w_spec(a) for a in weights]
    return pl.pallas_call(
        _mlp_body,
        grid=(BATCH // BB,),
        in_specs=in_specs,
        out_specs=pl.BlockSpec((BB, 2), lambda i: (i, 0)),
        out_shape=jax.ShapeDtypeStruct((BATCH, 2), jnp.float32),
    )(feat, *weights)


def kernel(x, emb_table,
           ctr_W0, ctr_b0, ctr_W1, ctr_b1, ctr_W2, ctr_b2, ctr_W3, ctr_b3, ctr_W4, ctr_b4,
           cvr_W0, cvr_b0, cvr_W1, cvr_b1, cvr_W2, cvr_b2, cvr_W3, cvr_b3, cvr_W4, cvr_b4):
    lin = _detranspose(emb_table.T)
    lin2d = lin.reshape(LIN_ROWS, PAD_DIM)
    idx = x.reshape(ROWS)
    feat = _gather(idx, lin2d).reshape(BATCH, IN_DIM_P)

    def pad_w0(W0):
        return jnp.zeros((IN_DIM_P, W0.shape[1]), W0.dtype).at[
            (jnp.arange(IN_DIM) // EMBED_DIM) * PAD_DIM + jnp.arange(IN_DIM) % EMBED_DIM
        ].set(W0)

    cs = [pad_w0(ctr_W0), ctr_b0, ctr_W1, ctr_b1, ctr_W2, ctr_b2, ctr_W3, ctr_b3, ctr_W4, ctr_b4]
    vs = [pad_w0(cvr_W0), cvr_b0, cvr_W1, cvr_b1, cvr_W2, cvr_b2, cvr_W3, cvr_b3, cvr_W4, cvr_b4]
    weights = [a if a.ndim == 2 else a.reshape(1, -1) for a in cs + vs]
    out = _mlp(feat, weights)
    return (out[:, 0:1], out[:, 1:2])


# speedup vs baseline: 2.4457x; 2.4457x over previous
import jax
import jax.numpy as jnp
from jax import lax
from jax.experimental import pallas as pl
from jax.experimental.pallas import tpu as pltpu
from jax.experimental.pallas import tpu_sc as plsc

EMBED_DIM = 18
PAD_DIM = 24
FIELDS = 26
BATCH = 4096
VOCAB = 1000000
IN_DIM = FIELDS * EMBED_DIM
IN_DIM_P = FIELDS * PAD_DIM

NUM_CORES = 2
NUM_SUBCORES = 16
NW = NUM_CORES * NUM_SUBCORES
ROWS = BATCH * FIELDS
RPW = ROWS // NW

NSLAB = (VOCAB + 127) // 128
SPW = (NSLAB + NW - 1) // NW
LIN_ROWS = NSLAB * 128
SLAB_WORDS = 128 * PAD_DIM
LIN_LEN = NSLAB * SLAB_WORDS


def _detr_body(tab_hbm, out_hbm, slab0, slab1, lin0, lin1, sem_i0, sem_i1, sem_o):
    wid = lax.axis_index("s") * NUM_CORES + lax.axis_index("c")
    d1 = lax.iota(jnp.int32, 16)
    bvecs = [(d1 + k) & 15 for k in range(16)]
    sbases = [b * PAD_DIM + d1 for b in bvecs]
    s2bases = [d1 * PAD_DIM + d for d in (16, 17)]

    def cidx(s):
        return jnp.minimum(s * NW + wid, NSLAB - 1)

    def start_in(buf, sem, s):
        c = cidx(s)
        pltpu.async_copy(
            tab_hbm.at[:, pl.ds(pl.multiple_of(c * 128, 128), 128)], buf, sem)

    def wait_in(buf, sem):
        pltpu.make_async_copy(tab_hbm.at[:, pl.ds(0, 128)], buf, sem).wait()

    def drain_out(lin):
        pltpu.make_async_copy(out_hbm.at[pl.ds(0, SLAB_WORDS)], lin, sem_o).wait()

    def transpose(buf, lin):
        for l0 in range(0, 128, 16):
            for k in range(16):
                lvec = bvecs[k] + l0
                v = plsc.load_gather(buf, [d1, lvec])
                plsc.store_scatter(lin, [sbases[k] + l0 * PAD_DIM], v)
            for t, d in enumerate((16, 17)):
                v = buf[d, pl.ds(l0, 16)]
                plsc.store_scatter(lin, [s2bases[t] + l0 * PAD_DIM], v)

    def start_out(lin, s):
        c = cidx(s)
        pltpu.async_copy(
            lin,
            out_hbm.at[pl.ds(pl.multiple_of(c * SLAB_WORDS, 1024), SLAB_WORDS)],
            sem_o)

    zeros16 = jnp.zeros((16,), jnp.float32)
    for lin in (lin0, lin1):
        for i in range(SLAB_WORDS // 16):
            lin[pl.ds(i * 16, 16)] = zeros16

    start_in(slab0, sem_i0, 0)

    def body(j, carry):
        s0 = 2 * j
        wait_in(slab0, sem_i0)
        start_in(slab1, sem_i1, s0 + 1)

        @pl.when(j > 0)
        def _():
            drain_out(lin0)

        transpose(slab0, lin0)
        start_out(lin0, s0)

        wait_in(slab1, sem_i1)
        start_in(slab0, sem_i0, s0 + 2)

        @pl.when(j > 0)
        def _():
            drain_out(lin1)

        transpose(slab1, lin1)
        start_out(lin1, s0 + 1)
        return carry

    lax.fori_loop(0, (SPW - 1) // 2, body, 0)
    wait_in(slab0, sem_i0)
    drain_out(lin0)
    transpose(slab0, lin0)
    start_out(lin0, SPW - 1)
    drain_out(lin1)
    drain_out(lin0)


_detranspose = pl.kernel(
    _detr_body,
    out_type=jax.ShapeDtypeStruct((LIN_LEN,), jnp.float32),
    mesh=plsc.VectorSubcoreMesh(core_axis_name="c", subcore_axis_name="s"),
    scratch_types=[
        pltpu.VMEM((EMBED_DIM, 128), jnp.float32),
        pltpu.VMEM((EMBED_DIM, 128), jnp.float32),
        pltpu.VMEM((SLAB_WORDS,), jnp.float32),
        pltpu.VMEM((SLAB_WORDS,), jnp.float32),
        pltpu.SemaphoreType.DMA,
        pltpu.SemaphoreType.DMA,
        pltpu.SemaphoreType.DMA,
    ],
    compiler_params=pltpu.CompilerParams(needs_layout_passes=False),
)


def _gather_body(idx_hbm, lin_hbm, out_hbm, idx_v, rows_v, sem):
    wid = lax.axis_index("s") * NUM_CORES + lax.axis_index("c")
    base = wid * RPW
    pltpu.sync_copy(idx_hbm.at[pl.ds(base, RPW)], idx_v)
    pltpu.async_copy(lin_hbm.at[idx_v], rows_v, sem).wait()
    pltpu.sync_copy(rows_v, out_hbm.at[pl.ds(base, RPW)])


_gather = pl.kernel(
    _gather_body,
    out_type=jax.ShapeDtypeStruct((ROWS, PAD_DIM), jnp.float32),
    mesh=plsc.VectorSubcoreMesh(core_axis_name="c", subcore_axis_name="s"),
    scratch_types=[
        pltpu.VMEM((RPW,), jnp.int32),
        pltpu.VMEM((RPW, PAD_DIM), jnp.float32),
        pltpu.SemaphoreType.DMA,
    ],
    compiler_params=pltpu.CompilerParams(use_tc_tiling_on_sc=False),
)

BB = 1024


def _mlp_body(feat_ref,
              cW0, cb0, cW1, cb1, cW2, cb2, cW3, cb3, cW4, cb4,
              vW0, vb0, vW1, vb1, vW2, vb2, vW3, vb3, vW4, vb4,
              out_ref):
    f = feat_ref[...]

    def tower(Ws, bs):
        h = f
        for i in range(4):
            h = jnp.dot(h, Ws[i][...], preferred_element_type=jnp.float32)
            h = jnp.maximum(h + bs[i][...], 0.0)
        h = jnp.dot(h, Ws[4][...], preferred_element_type=jnp.float32)
        return h + bs[4][...]

    ctr = tower([cW0, cW1, cW2, cW3, cW4], [cb0, cb1, cb2, cb3, cb4])
    cvr = tower([vW0, vW1, vW2, vW3, vW4], [vb0, vb1, vb2, vb3, vb4])
    both = jnp.concatenate([ctr, cvr], axis=1)
    out_ref[...] = 1.0 / (1.0 + jnp.exp(-both))


def _mlp(feat, weights):
    def w_spec(w):
        return pl.BlockSpec(w.shape, lambda i: (0,) * w.ndim)

    in_specs = [pl.BlockSpec((BB, IN_DIM_P), lambda i: (i, 0))]
    in_specs += [w_spec(a) for a in weights]
    return pl.pallas_call(
        _mlp_body,
        grid=(BATCH // BB,),
        in_specs=in_specs,
        out_specs=pl.BlockSpec((BB, 2), lambda i: (i, 0)),
        out_shape=jax.ShapeDtypeStruct((BATCH, 2), jnp.float32),
    )(feat, *weights)


def kernel(x, emb_table,
           ctr_W0, ctr_b0, ctr_W1, ctr_b1, ctr_W2, ctr_b2, ctr_W3, ctr_b3, ctr_W4, ctr_b4,
           cvr_W0, cvr_b0, cvr_W1, cvr_b1, cvr_W2, cvr_b2, cvr_W3, cvr_b3, cvr_W4, cvr_b4):
    lin = _detranspose(emb_table.T)
    lin2d = lin.reshape(LIN_ROWS, PAD_DIM)
    idx = x.reshape(ROWS)
    feat = _gather(idx, lin2d).reshape(BATCH, IN_DIM_P)

    def pad_w0(W0):
        return jnp.zeros((IN_DIM_P, W0.shape[1]), W0.dtype).at[
            (jnp.arange(IN_DIM) // EMBED_DIM) * PAD_DIM + jnp.arange(IN_DIM) % EMBED_DIM
        ].set(W0)

    cs = [pad_w0(ctr_W0), ctr_b0, ctr_W1, ctr_b1, ctr_W2, ctr_b2, ctr_W3, ctr_b3, ctr_W4, ctr_b4]
    vs = [pad_w0(cvr_W0), cvr_b0, cvr_W1, cvr_b1, cvr_W2, cvr_b2, cvr_W3, cvr_b3, cvr_W4, cvr_b4]
    weights = [a if a.ndim == 2 else a.reshape(1, -1) for a in cs + vs]
    out = _mlp(feat, weights)
    return (out[:, 0:1], out[:, 1:2])


# speedup vs baseline: 2.7134x; 1.1094x over previous
import jax
import jax.numpy as jnp
from jax import lax
from jax.experimental import pallas as pl
from jax.experimental.pallas import tpu as pltpu
from jax.experimental.pallas import tpu_sc as plsc

EMBED_DIM = 18
PAD_DIM = 24
FIELDS = 26
BATCH = 4096
VOCAB = 1000000
IN_DIM = FIELDS * EMBED_DIM
IN_DIM_P = FIELDS * PAD_DIM

NUM_CORES = 2
NUM_SUBCORES = 16
NW = NUM_CORES * NUM_SUBCORES
ROWS = BATCH * FIELDS
RPW = ROWS // NW

NSLAB = (VOCAB + 127) // 128
SPW = (NSLAB + NW - 1) // NW
LIN_ROWS = NSLAB * 128
SLAB_WORDS = 128 * PAD_DIM
LIN_LEN = NSLAB * SLAB_WORDS


def _detr_body(tab_hbm, out_hbm, slab0, slab1, lin0, lin1, sem_i0, sem_i1, sem_o):
    wid = lax.axis_index("s") * NUM_CORES + lax.axis_index("c")
    d1 = lax.iota(jnp.int32, 16)
    bvecs = [(d1 + k) & 15 for k in range(16)]
    sbases = [b * PAD_DIM + d1 for b in bvecs]
    s2bases = [d1 * PAD_DIM + d for d in (16, 17)]

    def cidx(s):
        return jnp.minimum(s * NW + wid, NSLAB - 1)

    def start_in(buf, sem, s):
        c = cidx(s)
        pltpu.async_copy(
            tab_hbm.at[:, pl.ds(pl.multiple_of(c * 128, 128), 128)], buf, sem)

    def wait_in(buf, sem):
        pltpu.make_async_copy(tab_hbm.at[:, pl.ds(0, 128)], buf, sem).wait()

    def drain_out(lin):
        pltpu.make_async_copy(out_hbm.at[pl.ds(0, SLAB_WORDS)], lin, sem_o).wait()

    def transpose(buf, lin):
        for l0 in range(0, 128, 16):
            for k0 in range(0, 16, 8):
                vs = [plsc.load_gather(buf, [d1, bvecs[k] + l0])
                      for k in range(k0, k0 + 8)]
                for k, v in zip(range(k0, k0 + 8), vs):
                    plsc.store_scatter(lin, [sbases[k] + l0 * PAD_DIM], v)
            v16 = buf[16, pl.ds(l0, 16)]
            v17 = buf[17, pl.ds(l0, 16)]
            plsc.store_scatter(lin, [s2bases[0] + l0 * PAD_DIM], v16)
            plsc.store_scatter(lin, [s2bases[1] + l0 * PAD_DIM], v17)

    def start_out(lin, s):
        c = cidx(s)
        pltpu.async_copy(
            lin,
            out_hbm.at[pl.ds(pl.multiple_of(c * SLAB_WORDS, 1024), SLAB_WORDS)],
            sem_o)

    zeros16 = jnp.zeros((16,), jnp.float32)
    for lin in (lin0, lin1):
        for i in range(SLAB_WORDS // 16):
            lin[pl.ds(i * 16, 16)] = zeros16

    start_in(slab0, sem_i0, 0)

    def body(j, carry):
        s0 = 2 * j
        wait_in(slab0, sem_i0)
        start_in(slab1, sem_i1, s0 + 1)

        @pl.when(j > 0)
        def _():
            drain_out(lin0)

        transpose(slab0, lin0)
        start_out(lin0, s0)

        wait_in(slab1, sem_i1)
        start_in(slab0, sem_i0, s0 + 2)

        @pl.when(j > 0)
        def _():
            drain_out(lin1)

        transpose(slab1, lin1)
        start_out(lin1, s0 + 1)
        return carry

    lax.fori_loop(0, (SPW - 1) // 2, body, 0)
    wait_in(slab0, sem_i0)
    drain_out(lin0)
    transpose(slab0, lin0)
    start_out(lin0, SPW - 1)
    drain_out(lin1)
    drain_out(lin0)


_detranspose = pl.kernel(
    _detr_body,
    out_type=jax.ShapeDtypeStruct((LIN_LEN,), jnp.float32),
    mesh=plsc.VectorSubcoreMesh(core_axis_name="c", subcore_axis_name="s"),
    scratch_types=[
        pltpu.VMEM((EMBED_DIM, 128), jnp.float32),
        pltpu.VMEM((EMBED_DIM, 128), jnp.float32),
        pltpu.VMEM((SLAB_WORDS,), jnp.float32),
        pltpu.VMEM((SLAB_WORDS,), jnp.float32),
        pltpu.SemaphoreType.DMA,
        pltpu.SemaphoreType.DMA,
        pltpu.SemaphoreType.DMA,
    ],
    compiler_params=pltpu.CompilerParams(needs_layout_passes=False),
)


def _gather_body(idx_hbm, lin_hbm, out_hbm, idx_v, rows_v, sem):
    wid = lax.axis_index("s") * NUM_CORES + lax.axis_index("c")
    base = wid * RPW
    pltpu.sync_copy(idx_hbm.at[pl.ds(base, RPW)], idx_v)
    pltpu.async_copy(lin_hbm.at[idx_v], rows_v, sem).wait()
    pltpu.sync_copy(rows_v, out_hbm.at[pl.ds(base, RPW)])


_gather = pl.kernel(
    _gather_body,
    out_type=jax.ShapeDtypeStruct((ROWS, PAD_DIM), jnp.float32),
    mesh=plsc.VectorSubcoreMesh(core_axis_name="c", subcore_axis_name="s"),
    scratch_types=[
        pltpu.VMEM((RPW,), jnp.int32),
        pltpu.VMEM((RPW, PAD_DIM), jnp.float32),
        pltpu.SemaphoreType.DMA,
    ],
    compiler_params=pltpu.CompilerParams(use_tc_tiling_on_sc=False),
)

BB = 1024


def _mlp_body(feat_ref,
              cW0, cb0, cW1, cb1, cW2, cb2, cW3, cb3, cW4, cb4,
              vW0, vb0, vW1, vb1, vW2, vb2, vW3, vb3, vW4, vb4,
              out_ref):
    f = feat_ref[...]

    def tower(Ws, bs):
        h = f
        for i in range(4):
            h = jnp.dot(h, Ws[i][...], preferred_element_type=jnp.float32)
            h = jnp.maximum(h + bs[i][...], 0.0)
        h = jnp.dot(h, Ws[4][...], preferred_element_type=jnp.float32)
        return h + bs[4][...]

    ctr = tower([cW0, cW1, cW2, cW3, cW4], [cb0, cb1, cb2, cb3, cb4])
    cvr = tower([vW0, vW1, vW2, vW3, vW4], [vb0, vb1, vb2, vb3, vb4])
    both = jnp.concatenate([ctr, cvr], axis=1)
    out_ref[...] = 1.0 / (1.0 + jnp.exp(-both))


def _mlp(feat, weights):
    def w_spec(w):
        return pl.BlockSpec(w.shape, lambda i: (0,) * w.ndim)

    in_specs = [pl.BlockSpec((BB, IN_DIM_P), lambda i: (i, 0))]
    in_specs += [w_spec(a) for a in weights]
    return pl.pallas_call(
        _mlp_body,
        grid=(BATCH // BB,),
        in_specs=in_specs,
        out_specs=pl.BlockSpec((BB, 2), lambda i: (i, 0)),
        out_shape=jax.ShapeDtypeStruct((BATCH, 2), jnp.float32),
    )(feat, *weights)


def kernel(x, emb_table,
           ctr_W0, ctr_b0, ctr_W1, ctr_b1, ctr_W2, ctr_b2, ctr_W3, ctr_b3, ctr_W4, ctr_b4,
           cvr_W0, cvr_b0, cvr_W1, cvr_b1, cvr_W2, cvr_b2, cvr_W3, cvr_b3, cvr_W4, cvr_b4):
    lin = _detranspose(emb_table.T)
    lin2d = lin.reshape(LIN_ROWS, PAD_DIM)
    idx = x.reshape(ROWS)
    feat = _gather(idx, lin2d).reshape(BATCH, IN_DIM_P)

    def pad_w0(W0):
        return jnp.zeros((IN_DIM_P, W0.shape[1]), W0.dtype).at[
            (jnp.arange(IN_DIM) // EMBED_DIM) * PAD_DIM + jnp.arange(IN_DIM) % EMBED_DIM
        ].set(W0)

    cs = [pad_w0(ctr_W0), ctr_b0, ctr_W1, ctr_b1, ctr_W2, ctr_b2, ctr_W3, ctr_b3, ctr_W4, ctr_b4]
    vs = [pad_w0(cvr_W0), cvr_b0, cvr_W1, cvr_b1, cvr_W2, cvr_b2, cvr_W3, cvr_b3, cvr_W4, cvr_b4]
    weights = [a if a.ndim == 2 else a.reshape(1, -1) for a in cs + vs]
    out = _mlp(feat, weights)
    return (out[:, 0:1], out[:, 1:2])
